# baseline (speedup 1.0000x reference)
import functools

import jax
import jax.numpy as jnp
from jax import lax
from jax.experimental import pallas as pl
from jax.experimental.pallas import tpu as pltpu
from jax.experimental.pallas import tpu_sc as plsc

_NUM_CORES = 2
_NUM_SUBCORES = 16
_NUM_WORKERS = _NUM_CORES * _NUM_SUBCORES
_CHUNK_B = 8


@functools.lru_cache(maxsize=None)
def _make_gather(Bm, T, Tp, D):
    bw = Bm // _NUM_WORKERS
    n_chunks = bw // _CHUNK_B
    mesh = plsc.VectorSubcoreMesh(core_axis_name="c", subcore_axis_name="s")

    @functools.partial(
        pl.kernel,
        mesh=mesh,
        out_type=jax.ShapeDtypeStruct((Bm, T, D), jnp.float32),
        compiler_params=pltpu.CompilerParams(use_tc_tiling_on_sc=False),
        scratch_types=[
            pltpu.VMEM((bw * Tp,), jnp.int32),
            pltpu.VMEM((2, _CHUNK_B, Tp, D), jnp.float32),
            pltpu.SemaphoreType.DMA,
            pltpu.SemaphoreType.DMA,
            pltpu.SemaphoreType.DMA,
            pltpu.SemaphoreType.DMA,
        ],
    )
    def gather_kernel(idx_hbm, table_hbm, out_hbm, idx_v, rows_v,
                      gsem0, gsem1, ssem0, ssem1):
        wid = lax.axis_index("s") * _NUM_CORES + lax.axis_index("c")
        b0 = wid * bw
        gsem = (gsem0, gsem1)
        ssem = (ssem0, ssem1)

        pltpu.sync_copy(idx_hbm.at[pl.ds(b0 * Tp, bw * Tp)], idx_v)

        def fire_gathers(g, b):
            handles = []
            for j in range(_CHUNK_B):
                handles.append(pltpu.async_copy(
                    table_hbm.at[idx_v.at[pl.ds((g * _CHUNK_B + j) * Tp, Tp)]],
                    rows_v.at[b, j], gsem[b]))
            return handles

        gathers = [None] * n_chunks
        stores = [None] * n_chunks
        for g in range(n_chunks):
            b = g % 2
            if g >= 2:
                stores[g - 2].wait()
            gathers[g] = fire_gathers(g, b)
            if g >= 1:
                for h in gathers[g - 1]:
                    h.wait()
                stores[g - 1] = pltpu.async_copy(
                    rows_v.at[(g - 1) % 2, :, pl.ds(0, T)],
                    out_hbm.at[pl.ds(b0 + (g - 1) * _CHUNK_B, _CHUNK_B)],
                    ssem[(g - 1) % 2])
        g = n_chunks - 1
        for h in gathers[g]:
            h.wait()
        stores[g] = pltpu.async_copy(
            rows_v.at[g % 2, :, pl.ds(0, T)],
            out_hbm.at[pl.ds(b0 + g * _CHUNK_B, _CHUNK_B)], ssem[g % 2])
        stores[g - 1].wait()
        stores[g].wait()

    return gather_kernel


def kernel(input, weight):
    Bm, T = input.shape
    D = weight.shape[1]
    Tp = (T + 7) // 8 * 8
    idx = jnp.pad(input.astype(jnp.int32), ((0, 0), (0, Tp - T)))
    return _make_gather(Bm, T, Tp, D)(idx.reshape(Bm * Tp), weight)


# speedup vs baseline: 2.0717x; 2.0717x over previous
import functools

import jax
import jax.numpy as jnp
from jax import lax
from jax.experimental import pallas as pl
from jax.experimental.pallas import tpu as pltpu
from jax.experimental.pallas import tpu_sc as plsc

_NUM_CORES = 2
_NUM_SUBCORES = 16
_NUM_WORKERS = _NUM_CORES * _NUM_SUBCORES
_CHUNK_B = 32


@functools.lru_cache(maxsize=None)
def _make_gather(Bm, T, D):
    bw = Bm // _NUM_WORKERS
    n_chunks = bw // _CHUNK_B
    cidx = _CHUNK_B * T
    mesh = plsc.VectorSubcoreMesh(core_axis_name="c", subcore_axis_name="s")

    @functools.partial(
        pl.kernel,
        mesh=mesh,
        out_type=jax.ShapeDtypeStruct((Bm, T, D), jnp.float32),
        compiler_params=pltpu.CompilerParams(use_tc_tiling_on_sc=False),
        scratch_types=[
            pltpu.VMEM((bw * T,), jnp.int32),
            pltpu.VMEM((2, cidx, D), jnp.float32),
            pltpu.SemaphoreType.DMA,
            pltpu.SemaphoreType.DMA,
            pltpu.SemaphoreType.DMA,
            pltpu.SemaphoreType.DMA,
        ],
    )
    def gather_kernel(idx_hbm, table_hbm, out_hbm, idx_v, rows_v,
                      gsem0, gsem1, ssem0, ssem1):
        wid = lax.axis_index("s") * _NUM_CORES + lax.axis_index("c")
        b0 = wid * bw
        gsem = (gsem0, gsem1)
        ssem = (ssem0, ssem1)

        pltpu.sync_copy(idx_hbm.at[pl.ds(b0 * T, bw * T)], idx_v)

        def fire_stores(g, b):
            handles = []
            for j in range(_CHUNK_B):
                handles.append(pltpu.async_copy(
                    rows_v.at[b, pl.ds(j * T, T)],
                    out_hbm.at[b0 + g * _CHUNK_B + j], ssem[b]))
            return handles

        gathers = [None] * n_chunks
        stores = [None] * n_chunks
        for g in range(n_chunks):
            b = g % 2
            if g >= 2:
                for h in stores[g - 2]:
                    h.wait()
            gathers[g] = pltpu.async_copy(
                table_hbm.at[idx_v.at[pl.ds(g * cidx, cidx)]],
                rows_v.at[b], gsem[b])
            if g >= 1:
                gathers[g - 1].wait()
                stores[g - 1] = fire_stores(g - 1, (g - 1) % 2)
        g = n_chunks - 1
        gathers[g].wait()
        stores[g] = fire_stores(g, g % 2)
        for h in stores[g - 1]:
            h.wait()
        for h in stores[g]:
            h.wait()

    return gather_kernel


def kernel(input, weight):
    Bm, T = input.shape
    D = weight.shape[1]
    idx = input.reshape(Bm * T).astype(jnp.int32)
    return _make_gather(Bm, T, D)(idx, weight)


# speedup vs baseline: 2.2293x; 1.0761x over previous
import functools

import jax
import jax.numpy as jnp
from jax import lax
from jax.experimental import pallas as pl
from jax.experimental.pallas import tpu as pltpu
from jax.experimental.pallas import tpu_sc as plsc

_NUM_CORES = 2
_NUM_SUBCORES = 16
_NUM_WORKERS = _NUM_CORES * _NUM_SUBCORES
_CHUNK = 1280


@functools.lru_cache(maxsize=None)
def _make_gather(B, D):
    b_per_w = B // _NUM_WORKERS
    n_chunks = b_per_w // _CHUNK
    mesh = plsc.VectorSubcoreMesh(core_axis_name="c", subcore_axis_name="s")

    @functools.partial(
        pl.kernel,
        mesh=mesh,
        out_type=jax.ShapeDtypeStruct((B, D), jnp.float32),
        compiler_params=pltpu.CompilerParams(use_tc_tiling_on_sc=False),
        scratch_types=[
            pltpu.VMEM((b_per_w,), jnp.int32),
            pltpu.VMEM((2, _CHUNK, D), jnp.float32),
            pltpu.SemaphoreType.DMA,
            pltpu.SemaphoreType.DMA,
            pltpu.SemaphoreType.DMA,
            pltpu.SemaphoreType.DMA,
        ],
    )
    def gather_kernel(idx_hbm, table_hbm, out_hbm, idx_v, rows_v,
                      gsem0, gsem1, ssem0, ssem1):
        wid = lax.axis_index("s") * _NUM_CORES + lax.axis_index("c")
        base = wid * b_per_w
        gsem = (gsem0, gsem1)
        ssem = (ssem0, ssem1)

        pltpu.sync_copy(idx_hbm.at[pl.ds(base, b_per_w)], idx_v)

        gathers = [None] * n_chunks
        stores = [None] * n_chunks
        for g in range(n_chunks):
            b = g % 2
            if g >= 2:
                stores[g - 2].wait()
            gathers[g] = pltpu.async_copy(
                table_hbm.at[idx_v.at[pl.ds(g * _CHUNK, _CHUNK)]],
                rows_v.at[b], gsem[b])
            if g >= 1:
                gathers[g - 1].wait()
                stores[g - 1] = pltpu.async_copy(
                    rows_v.at[(g - 1) % 2],
                    out_hbm.at[pl.ds(base + (g - 1) * _CHUNK, _CHUNK)],
                    ssem[(g - 1) % 2])
        g = n_chunks - 1
        gathers[g].wait()
        stores[g] = pltpu.async_copy(
            rows_v.at[g % 2],
            out_hbm.at[pl.ds(base + g * _CHUNK, _CHUNK)], ssem[g % 2])
        stores[g - 1].wait()
        stores[g].wait()

    return gather_kernel


def kernel(input, weight):
    Bm, T = input.shape
    D = weight.shape[1]
    B = Bm * T
    idx = jnp.transpose(input).reshape(B).astype(jnp.int32)
    out = _make_gather(B, D)(idx, weight)
    return jnp.transpose(out.reshape(T, Bm, D), (1, 0, 2))
